# baseline (speedup 1.0000x reference)
import functools

import jax
import jax.numpy as jnp
from jax import lax
from jax.experimental import pallas as pl
from jax.experimental.pallas import tpu as pltpu
from jax.experimental.pallas import tpu_sc as plsc

N = 10000
NP = 10240
K = 16
BIG = 3.0e38
PADVAL = 1.0e18

SC_CORES = 2
SC_SUBCORES = 16
NW = SC_CORES * SC_SUBCORES
BPW = NP // NW


def _coords_body(x_ref, w_ref, b_ref, o_ref):
    i = pl.program_id(0)
    v = jnp.dot(x_ref[...], w_ref[...], preferred_element_type=jnp.float32)
    v = v + b_ref[...]
    rows = lax.broadcasted_iota(jnp.int32, v.shape, 0) + i * v.shape[0]
    v = jnp.where(rows >= N, PADVAL, v)
    n2 = (v[:, 0:1] * v[:, 0:1] + v[:, 1:2] * v[:, 1:2]
          + v[:, 2:3] * v[:, 2:3])
    o_ref[...] = jnp.concatenate([v[:, 0:6], n2, v[:, 7:8]], axis=1)


def _stage1(x_pad, Wsh, bsh):
    B = 1024
    return pl.pallas_call(
        _coords_body,
        grid=(NP // B,),
        in_specs=[
            pl.BlockSpec((B, 128), lambda i: (i, 0)),
            pl.BlockSpec((128, 8), lambda i: (0, 0)),
            pl.BlockSpec((1, 8), lambda i: (0, 0)),
        ],
        out_specs=pl.BlockSpec((B, 8), lambda i: (i, 0)),
        out_shape=jax.ShapeDtypeStruct((NP, 8), jnp.float32),
    )(x_pad, Wsh, bsh)


TL = 1024
NT = NP // TL


def _knn_body(sr_ref, st_ref, idx_ref, w_ref, dscr):
    sb = sr_ref[...]
    B = sb.shape[0]

    lhs = jnp.concatenate(
        [sb[:, 0:3] * (-2.0), sb[:, 6:7], jnp.ones((B, 1), jnp.float32),
         jnp.zeros((B, 3), jnp.float32)], axis=1)

    def fill(t, carry):
        st_t = st_ref[:, pl.ds(t * TL, TL)]
        dscr[:, pl.ds(t * TL, TL)] = jax.lax.dot_general(
            lhs, st_t, (((1,), (0,)), ((), ())),
            precision=jax.lax.Precision.HIGHEST,
            preferred_element_type=jnp.float32)
        return carry

    lax.fori_loop(0, NT, fill, 0)

    kcol = lax.broadcasted_iota(jnp.int32, (B, K), 1)
    lane = lax.broadcasted_iota(jnp.int32, (B, 128), 1)
    NG = TL // 128

    def select(k, prev):
        pm, m_acc, i_acc = prev
        pm_b = jnp.broadcast_to(pm, (B, 128))

        def scan(t, carry):
            bv, bg = carry
            for g in range(NG):
                dd = dscr[:, pl.ds(t * TL + g * 128, 128)]
                upd = (dd > pm_b) & (dd < bv)
                bv = jnp.where(upd, dd, bv)
                bg = jnp.where(upd, t * NG + g, bg)
            return bv, bg

        bv, bg = lax.fori_loop(
            0, NT, scan,
            (jnp.full((B, 128), BIG, jnp.float32),
             jnp.zeros((B, 128), jnp.int32)),
            unroll=NT)
        ci = bg * 128 + lane
        m = jnp.min(bv, axis=1, keepdims=True)
        isel = jnp.min(jnp.where(bv == m, ci, NP), axis=1, keepdims=True)
        hit = kcol == k
        return (m, jnp.where(hit, m, m_acc), jnp.where(hit, isel, i_acc))

    _, m_acc, i_acc = lax.fori_loop(
        0, K, select,
        (jnp.full((B, 1), -1.0, jnp.float32),
         jnp.zeros((B, K), jnp.float32), jnp.zeros((B, K), jnp.int32)))
    idx_ref[...] = i_acc
    w_ref[...] = jnp.exp(-10.0 * m_acc)

    t16 = jnp.broadcast_to(m_acc[:, K - 1:K], (B, 128))

    def count(t, c):
        for g in range(NG):
            dd = dscr[:, pl.ds(t * TL + g * 128, 128)]
            c = c + jnp.where(dd <= t16, 1, 0)
        return c

    cnt = lax.fori_loop(0, NT, count, jnp.zeros((B, 128), jnp.int32),
                        unroll=NT)
    bad = jnp.max(jnp.abs(jnp.sum(cnt, axis=1, keepdims=True) - K))

    @pl.when(bad > 0)
    def _fallback():
        def select2(k, prev):
            pm, pi, m_acc, i_acc = prev

            def scan2(t, carry):
                bv, bi = carry
                dd = dscr[:, pl.ds(t * TL, TL)]
                ii = lax.broadcasted_iota(jnp.int32, (B, TL), 1) + t * TL
                cond = (dd > pm) | ((dd == pm) & (ii > pi))
                dm = jnp.where(cond, dd, BIG)
                m_t = jnp.min(dm, axis=1, keepdims=True)
                i_t = jnp.min(jnp.where(dm == m_t, ii, NP), axis=1,
                              keepdims=True)
                upd = (m_t < bv) | ((m_t == bv) & (i_t < bi))
                return jnp.where(upd, m_t, bv), jnp.where(upd, i_t, bi)

            bv, bi = lax.fori_loop(
                0, NT, scan2,
                (jnp.full((B, 1), BIG, jnp.float32), jnp.full((B, 1), NP)))
            hit = kcol == k
            return (bv, bi, jnp.where(hit, bv, m_acc),
                    jnp.where(hit, bi, i_acc))

        _, _, m2, i2 = lax.fori_loop(
            0, K, select2,
            (jnp.full((B, 1), -1.0, jnp.float32), jnp.full((B, 1), -1),
             jnp.zeros((B, K), jnp.float32), jnp.zeros((B, K), jnp.int32)))
        idx_ref[...] = i2
        w_ref[...] = jnp.exp(-10.0 * m2)


def _stage2(sh, sh_t):
    B = 128
    return pl.pallas_call(
        _knn_body,
        grid=(NP // B,),
        in_specs=[
            pl.BlockSpec((B, 8), lambda i: (i, 0)),
            pl.BlockSpec((8, NP), lambda i: (0, 0)),
        ],
        out_specs=[
            pl.BlockSpec((B, K), lambda i: (i, 0)),
            pl.BlockSpec((B, K), lambda i: (i, 0)),
        ],
        out_shape=[
            jax.ShapeDtypeStruct((NP, K), jnp.int32),
            jax.ShapeDtypeStruct((NP, K), jnp.float32),
        ],
        scratch_shapes=[
            pltpu.VMEM((B, NP), jnp.float32),
        ],
    )(sh, sh_t)


def _agg_body(idx_hbm, w_hbm, h0_hbm, h1_hbm, h2_hbm,
              o0_hbm, o1_hbm, o2_hbm, o3_hbm, o4_hbm, o5_hbm,
              idx_v, w_v, h0_v, h1_v, h2_v,
              o0_v, o1_v, o2_v, o3_v, o4_v, o5_v):
    cid = lax.axis_index("c")
    sid = lax.axis_index("s")
    wid = sid * SC_CORES + cid
    base = wid * BPW
    pltpu.sync_copy(h0_hbm, h0_v)
    pltpu.sync_copy(h1_hbm, h1_v)
    pltpu.sync_copy(h2_hbm, h2_v)
    pltpu.sync_copy(idx_hbm.at[pl.ds(base, BPW)], idx_v)
    pltpu.sync_copy(w_hbm.at[pl.ds(base, BPW)], w_v)

    lanes = lax.iota(jnp.int32, K)
    outs = (o0_v, o1_v, o2_v, o3_v, o4_v, o5_v)

    def body(g, carry):
        accs = [jnp.zeros((K,), jnp.float32) for _ in range(6)]
        base_i = g * K
        for j in range(K):
            iv = idx_v[base_i + j]
            wv = w_v[base_i + j]
            g0 = plsc.load_gather(h0_v, [iv]) * wv
            g1 = plsc.load_gather(h1_v, [iv]) * wv
            g2 = plsc.load_gather(h2_v, [iv]) * wv
            vals = (jnp.sum(g0) * (1.0 / K), jnp.sum(g1) * (1.0 / K),
                    jnp.sum(g2) * (1.0 / K),
                    jnp.max(g0), jnp.max(g1), jnp.max(g2))
            sel = lanes == j
            accs = [jnp.where(sel, v, a) for v, a in zip(vals, accs)]
        for t in range(6):
            outs[t][pl.ds(base_i, K)] = accs[t]
        return carry

    lax.fori_loop(0, BPW // K, body, 0)
    pltpu.sync_copy(o0_v, o0_hbm.at[pl.ds(base, BPW)])
    pltpu.sync_copy(o1_v, o1_hbm.at[pl.ds(base, BPW)])
    pltpu.sync_copy(o2_v, o2_hbm.at[pl.ds(base, BPW)])
    pltpu.sync_copy(o3_v, o3_hbm.at[pl.ds(base, BPW)])
    pltpu.sync_copy(o4_v, o4_hbm.at[pl.ds(base, BPW)])
    pltpu.sync_copy(o5_v, o5_hbm.at[pl.ds(base, BPW)])


def _stage3(idx, w, h0, h1, h2):
    f32 = jnp.float32
    out_type = [jax.ShapeDtypeStruct((NP,), f32) for _ in range(6)]
    scratch = [
        pltpu.VMEM((BPW, K), jnp.int32),
        pltpu.VMEM((BPW, K), f32),
        pltpu.VMEM((NP,), f32),
        pltpu.VMEM((NP,), f32),
        pltpu.VMEM((NP,), f32),
    ] + [pltpu.VMEM((BPW,), f32) for _ in range(6)]
    mesh = plsc.VectorSubcoreMesh(core_axis_name="c", subcore_axis_name="s")
    fn = pl.kernel(
        _agg_body,
        out_type=out_type,
        mesh=mesh,
        scratch_types=scratch,
        compiler_params=pltpu.CompilerParams(needs_layout_passes=False),
    )
    return fn(idx, w, h0, h1, h2)


def _out_body(x_ref, wo1_ref, agg_ref, wo2_ref, bo2_ref, wb_ref, bb_ref,
              lat_ref, beta_ref):
    acc = jnp.dot(x_ref[...], wo1_ref[...], preferred_element_type=jnp.float32)
    acc = acc + jnp.dot(agg_ref[...], wo2_ref[...],
                        preferred_element_type=jnp.float32)
    acc = acc + bo2_ref[...]
    lat_ref[...] = acc
    z = jnp.sum(acc * wb_ref[...], axis=1, keepdims=True) + bb_ref[...]
    b = jax.nn.sigmoid(z)
    beta_ref[...] = jnp.clip(b, 1e-6, 1.0 - 1e-6)


def _stage4(x_pad, Wo1, agg, Wo2e, bo2, wbT, bb):
    B = 512
    return pl.pallas_call(
        _out_body,
        grid=(NP // B,),
        in_specs=[
            pl.BlockSpec((B, 128), lambda i: (i, 0)),
            pl.BlockSpec((128, 128), lambda i: (0, 0)),
            pl.BlockSpec((B, 8), lambda i: (i, 0)),
            pl.BlockSpec((8, 128), lambda i: (0, 0)),
            pl.BlockSpec((1, 128), lambda i: (0, 0)),
            pl.BlockSpec((1, 128), lambda i: (0, 0)),
            pl.BlockSpec((1, 1), lambda i: (0, 0)),
        ],
        out_specs=[
            pl.BlockSpec((B, 128), lambda i: (i, 0)),
            pl.BlockSpec((B, 1), lambda i: (i, 0)),
        ],
        out_shape=[
            jax.ShapeDtypeStruct((NP, 128), jnp.float32),
            jax.ShapeDtypeStruct((NP, 1), jnp.float32),
        ],
    )(x_pad, Wo1, agg, Wo2e, bo2, wbT, bb)


def kernel(x, Ws, bs, Wh, bh, Wo1, Wo2, bo2, Wb, bb):
    f32 = jnp.float32
    x_pad = jnp.pad(x.astype(f32), ((0, NP - N), (0, 0)))
    Wsh = jnp.pad(jnp.concatenate([Ws, Wh], axis=1), ((0, 0), (0, 2)))
    bsh = jnp.pad(jnp.concatenate([bs, bh]), (0, 2)).reshape(1, 8)

    sh = _stage1(x_pad, Wsh, bsh)
    sh_t = sh.T
    ones = jnp.ones((1, NP), f32)
    zeros = jnp.zeros((3, NP), f32)
    st_mm = jnp.concatenate(
        [sh_t[0:3], ones, sh_t[6:7], zeros], axis=0)

    idx, w = _stage2(sh, st_mm)

    h0 = sh_t[3]
    h1 = sh_t[4]
    h2 = sh_t[5]
    m0, m1, m2, x0, x1, x2 = _stage3(idx, w, h0, h1, h2)

    agg = jnp.stack([m0, m1, m2, x0, x1, x2], axis=1)
    agg = jnp.pad(agg, ((0, 0), (0, 2)))
    Wo2e = jnp.pad(Wo2, ((0, 2), (0, 0)))
    latent, beta = _stage4(
        x_pad, Wo1, agg, Wo2e, bo2.reshape(1, 128),
        Wb.reshape(1, 128), bb.reshape(1, 1))
    return beta[:N, 0], latent[:N]


# speedup vs baseline: 1.1540x; 1.1540x over previous
import functools

import jax
import jax.numpy as jnp
from jax import lax
from jax.experimental import pallas as pl
from jax.experimental.pallas import tpu as pltpu
from jax.experimental.pallas import tpu_sc as plsc

N = 10000
NP = 10240
K = 16
BIG = 3.0e38
PADVAL = 1.0e18

SC_CORES = 2
SC_SUBCORES = 16
NW = SC_CORES * SC_SUBCORES
BPW = NP // NW


def _coords_body(x_ref, w_ref, b_ref, o_ref):
    i = pl.program_id(0)
    v = jnp.dot(x_ref[...], w_ref[...], preferred_element_type=jnp.float32)
    v = v + b_ref[...]
    rows = lax.broadcasted_iota(jnp.int32, v.shape, 0) + i * v.shape[0]
    v = jnp.where(rows >= N, PADVAL, v)
    n2 = (v[:, 0:1] * v[:, 0:1] + v[:, 1:2] * v[:, 1:2]
          + v[:, 2:3] * v[:, 2:3])
    o_ref[...] = jnp.concatenate([v[:, 0:6], n2, v[:, 7:8]], axis=1)


def _stage1(x_pad, Wsh, bsh):
    B = 1024
    return pl.pallas_call(
        _coords_body,
        grid=(NP // B,),
        in_specs=[
            pl.BlockSpec((B, 128), lambda i: (i, 0)),
            pl.BlockSpec((128, 8), lambda i: (0, 0)),
            pl.BlockSpec((1, 8), lambda i: (0, 0)),
        ],
        out_specs=pl.BlockSpec((B, 8), lambda i: (i, 0)),
        out_shape=jax.ShapeDtypeStruct((NP, 8), jnp.float32),
    )(x_pad, Wsh, bsh)


TL = 1024
NT = NP // TL


def _knn_body(sr_ref, st_ref, idx_ref, w_ref, dscr):
    sb = sr_ref[...]
    B = sb.shape[0]

    def fill(t, carry):
        st_t = st_ref[:, pl.ds(t * TL, TL)]
        d0 = sb[:, 0:1] - st_t[0:1, :]
        d1 = sb[:, 1:2] - st_t[1:2, :]
        d2 = sb[:, 2:3] - st_t[2:3, :]
        dscr[:, pl.ds(t * TL, TL)] = (d0 * d0 + d1 * d1) + d2 * d2
        return carry

    lax.fori_loop(0, NT, fill, 0)

    kcol = lax.broadcasted_iota(jnp.int32, (B, K), 1)
    lane = lax.broadcasted_iota(jnp.int32, (B, 128), 1)
    NG = TL // 128

    def select(k, prev):
        pm, m_acc, i_acc = prev
        pm_b = jnp.broadcast_to(pm, (B, 128))

        def scan(t, carry):
            bv, bg = carry
            for g in range(NG):
                dd = dscr[:, pl.ds(t * TL + g * 128, 128)]
                dm = jnp.where(dd > pm_b, dd, BIG)
                upd = dm < bv
                bv = jnp.where(upd, dm, bv)
                bg = jnp.where(upd, t * NG + g, bg)
            return bv, bg

        bv, bg = lax.fori_loop(
            0, NT, scan,
            (jnp.full((B, 128), BIG, jnp.float32),
             jnp.zeros((B, 128), jnp.int32)),
            unroll=NT)
        ci = bg * 128 + lane
        m = jnp.min(bv, axis=1, keepdims=True)
        isel = jnp.min(jnp.where(bv == m, ci, NP), axis=1, keepdims=True)
        hit = kcol == k
        return (m, jnp.where(hit, m, m_acc), jnp.where(hit, isel, i_acc))

    _, m_acc, i_acc = lax.fori_loop(
        0, K, select,
        (jnp.full((B, 1), -1.0, jnp.float32),
         jnp.zeros((B, K), jnp.float32), jnp.zeros((B, K), jnp.int32)))
    idx_ref[...] = i_acc
    w_ref[...] = jnp.exp(-10.0 * m_acc)

    t16 = jnp.broadcast_to(m_acc[:, K - 1:K], (B, 128))

    def count(t, c):
        for g in range(NG):
            dd = dscr[:, pl.ds(t * TL + g * 128, 128)]
            c = c + jnp.where(dd <= t16, 1, 0)
        return c

    cnt = lax.fori_loop(0, NT, count, jnp.zeros((B, 128), jnp.int32),
                        unroll=NT)
    bad = jnp.max(jnp.abs(jnp.sum(cnt, axis=1, keepdims=True) - K))

    @pl.when(bad > 0)
    def _fallback():
        def select2(k, prev):
            pm, pi, m_acc, i_acc = prev

            def scan2(t, carry):
                bv, bi = carry
                dd = dscr[:, pl.ds(t * TL, TL)]
                ii = lax.broadcasted_iota(jnp.int32, (B, TL), 1) + t * TL
                cond = (dd > pm) | ((dd == pm) & (ii > pi))
                dm = jnp.where(cond, dd, BIG)
                m_t = jnp.min(dm, axis=1, keepdims=True)
                i_t = jnp.min(jnp.where(dm == m_t, ii, NP), axis=1,
                              keepdims=True)
                upd = (m_t < bv) | ((m_t == bv) & (i_t < bi))
                return jnp.where(upd, m_t, bv), jnp.where(upd, i_t, bi)

            bv, bi = lax.fori_loop(
                0, NT, scan2,
                (jnp.full((B, 1), BIG, jnp.float32), jnp.full((B, 1), NP)))
            hit = kcol == k
            return (bv, bi, jnp.where(hit, bv, m_acc),
                    jnp.where(hit, bi, i_acc))

        _, _, m2, i2 = lax.fori_loop(
            0, K, select2,
            (jnp.full((B, 1), -1.0, jnp.float32), jnp.full((B, 1), -1),
             jnp.zeros((B, K), jnp.float32), jnp.zeros((B, K), jnp.int32)))
        idx_ref[...] = i2
        w_ref[...] = jnp.exp(-10.0 * m2)


def _stage2(sh, sh_t):
    B = 128
    return pl.pallas_call(
        _knn_body,
        grid=(NP // B,),
        in_specs=[
            pl.BlockSpec((B, 8), lambda i: (i, 0)),
            pl.BlockSpec((8, NP), lambda i: (0, 0)),
        ],
        out_specs=[
            pl.BlockSpec((B, K), lambda i: (i, 0)),
            pl.BlockSpec((B, K), lambda i: (i, 0)),
        ],
        out_shape=[
            jax.ShapeDtypeStruct((NP, K), jnp.int32),
            jax.ShapeDtypeStruct((NP, K), jnp.float32),
        ],
        scratch_shapes=[
            pltpu.VMEM((B, NP), jnp.float32),
        ],
    )(sh, sh_t)


def _agg_body(idx_hbm, w_hbm, h0_hbm, h1_hbm, h2_hbm,
              o0_hbm, o1_hbm, o2_hbm, o3_hbm, o4_hbm, o5_hbm,
              idx_v, w_v, h0_v, h1_v, h2_v,
              o0_v, o1_v, o2_v, o3_v, o4_v, o5_v):
    cid = lax.axis_index("c")
    sid = lax.axis_index("s")
    wid = sid * SC_CORES + cid
    base = wid * BPW
    pltpu.sync_copy(h0_hbm, h0_v)
    pltpu.sync_copy(h1_hbm, h1_v)
    pltpu.sync_copy(h2_hbm, h2_v)
    pltpu.sync_copy(idx_hbm.at[pl.ds(base, BPW)], idx_v)
    pltpu.sync_copy(w_hbm.at[pl.ds(base, BPW)], w_v)

    lanes = lax.iota(jnp.int32, K)
    outs = (o0_v, o1_v, o2_v, o3_v, o4_v, o5_v)

    def body(g, carry):
        accs = [jnp.zeros((K,), jnp.float32) for _ in range(6)]
        base_i = g * K
        for j in range(K):
            iv = idx_v[base_i + j]
            wv = w_v[base_i + j]
            g0 = plsc.load_gather(h0_v, [iv]) * wv
            g1 = plsc.load_gather(h1_v, [iv]) * wv
            g2 = plsc.load_gather(h2_v, [iv]) * wv
            vals = (jnp.sum(g0) * (1.0 / K), jnp.sum(g1) * (1.0 / K),
                    jnp.sum(g2) * (1.0 / K),
                    jnp.max(g0), jnp.max(g1), jnp.max(g2))
            sel = lanes == j
            accs = [jnp.where(sel, v, a) for v, a in zip(vals, accs)]
        for t in range(6):
            outs[t][pl.ds(base_i, K)] = accs[t]
        return carry

    lax.fori_loop(0, BPW // K, body, 0)
    pltpu.sync_copy(o0_v, o0_hbm.at[pl.ds(base, BPW)])
    pltpu.sync_copy(o1_v, o1_hbm.at[pl.ds(base, BPW)])
    pltpu.sync_copy(o2_v, o2_hbm.at[pl.ds(base, BPW)])
    pltpu.sync_copy(o3_v, o3_hbm.at[pl.ds(base, BPW)])
    pltpu.sync_copy(o4_v, o4_hbm.at[pl.ds(base, BPW)])
    pltpu.sync_copy(o5_v, o5_hbm.at[pl.ds(base, BPW)])


def _stage3(idx, w, h0, h1, h2):
    f32 = jnp.float32
    out_type = [jax.ShapeDtypeStruct((NP,), f32) for _ in range(6)]
    scratch = [
        pltpu.VMEM((BPW, K), jnp.int32),
        pltpu.VMEM((BPW, K), f32),
        pltpu.VMEM((NP,), f32),
        pltpu.VMEM((NP,), f32),
        pltpu.VMEM((NP,), f32),
    ] + [pltpu.VMEM((BPW,), f32) for _ in range(6)]
    mesh = plsc.VectorSubcoreMesh(core_axis_name="c", subcore_axis_name="s")
    fn = pl.kernel(
        _agg_body,
        out_type=out_type,
        mesh=mesh,
        scratch_types=scratch,
        compiler_params=pltpu.CompilerParams(needs_layout_passes=False),
    )
    return fn(idx, w, h0, h1, h2)


def _out_body(x_ref, wo1_ref, agg_ref, wo2_ref, bo2_ref, wb_ref, bb_ref,
              lat_ref, beta_ref):
    acc = jnp.dot(x_ref[...], wo1_ref[...], preferred_element_type=jnp.float32)
    acc = acc + jnp.dot(agg_ref[...], wo2_ref[...],
                        preferred_element_type=jnp.float32)
    acc = acc + bo2_ref[...]
    lat_ref[...] = acc
    z = jnp.sum(acc * wb_ref[...], axis=1, keepdims=True) + bb_ref[...]
    b = jax.nn.sigmoid(z)
    beta_ref[...] = jnp.clip(b, 1e-6, 1.0 - 1e-6)


def _stage4(x_pad, Wo1, agg, Wo2e, bo2, wbT, bb):
    B = 512
    return pl.pallas_call(
        _out_body,
        grid=(NP // B,),
        in_specs=[
            pl.BlockSpec((B, 128), lambda i: (i, 0)),
            pl.BlockSpec((128, 128), lambda i: (0, 0)),
            pl.BlockSpec((B, 8), lambda i: (i, 0)),
            pl.BlockSpec((8, 128), lambda i: (0, 0)),
            pl.BlockSpec((1, 128), lambda i: (0, 0)),
            pl.BlockSpec((1, 128), lambda i: (0, 0)),
            pl.BlockSpec((1, 1), lambda i: (0, 0)),
        ],
        out_specs=[
            pl.BlockSpec((B, 128), lambda i: (i, 0)),
            pl.BlockSpec((B, 1), lambda i: (i, 0)),
        ],
        out_shape=[
            jax.ShapeDtypeStruct((NP, 128), jnp.float32),
            jax.ShapeDtypeStruct((NP, 1), jnp.float32),
        ],
    )(x_pad, Wo1, agg, Wo2e, bo2, wbT, bb)


def kernel(x, Ws, bs, Wh, bh, Wo1, Wo2, bo2, Wb, bb):
    f32 = jnp.float32
    x_pad = jnp.pad(x.astype(f32), ((0, NP - N), (0, 0)))
    Wsh = jnp.pad(jnp.concatenate([Ws, Wh], axis=1), ((0, 0), (0, 2)))
    bsh = jnp.pad(jnp.concatenate([bs, bh]), (0, 2)).reshape(1, 8)

    sh = _stage1(x_pad, Wsh, bsh)
    sh_t = sh.T
    ones = jnp.ones((1, NP), f32)
    zeros = jnp.zeros((3, NP), f32)
    st_mm = jnp.concatenate(
        [sh_t[0:3], ones, sh_t[6:7], zeros], axis=0)

    idx, w = _stage2(sh, st_mm)

    h0 = sh_t[3]
    h1 = sh_t[4]
    h2 = sh_t[5]
    m0, m1, m2, x0, x1, x2 = _stage3(idx, w, h0, h1, h2)

    agg = jnp.stack([m0, m1, m2, x0, x1, x2], axis=1)
    agg = jnp.pad(agg, ((0, 0), (0, 2)))
    Wo2e = jnp.pad(Wo2, ((0, 2), (0, 0)))
    latent, beta = _stage4(
        x_pad, Wo1, agg, Wo2e, bo2.reshape(1, 128),
        Wb.reshape(1, 128), bb.reshape(1, 1))
    return beta[:N, 0], latent[:N]
